# baseline (speedup 1.0000x reference)
import functools

import jax
import jax.numpy as jnp
from jax import lax
from jax.experimental import pallas as pl
from jax.experimental.pallas import tpu as pltpu
from jax.experimental.pallas import tpu_sc as plsc

_N_NODES = 10000
_D = 256
_E = 160000
_E2 = 2 * _E
_NC, _NS, _L = 2, 16, 16
_NW = _NC * _NS
_EW = _E2 // _NW
_C = 80
_NCHUNK = _EW // _C
_DW = _D // 2
_CB = 25
_BE = _CB * _C


def _sc_dots_body(z_hbm, ps_hbm, pd_hbm, ns_hbm, nd_hbm, out_hbm,
                  z_sp, idx_s, idx_d, rows_s, rows_d, dots_v, red_v,
                  sem0, sem1):
    sid = lax.axis_index("s")
    wid = sid * _NC + lax.axis_index("c")
    base_w = wid * _EW

    rpt = _N_NODES // _NS
    pltpu.sync_copy(z_hbm.at[pl.ds(sid * rpt, rpt)],
                    z_sp.at[pl.ds(sid * rpt, rpt)])
    plsc.subcore_barrier()

    def stage_idx(blk):
        b = blk % 2

        @pl.when(wid < _NW // 2)
        def _():
            off = base_w + blk * _BE
            pltpu.sync_copy(ps_hbm.at[pl.ds(off, _BE)], idx_s.at[b])
            pltpu.sync_copy(pd_hbm.at[pl.ds(off, _BE)], idx_d.at[b])

        @pl.when(wid >= _NW // 2)
        def _():
            off = base_w - _E + blk * _BE
            pltpu.sync_copy(ns_hbm.at[pl.ds(off, _BE)], idx_s.at[b])
            pltpu.sync_copy(nd_hbm.at[pl.ds(off, _BE)], idx_d.at[b])

    def issue(ci, rs, rd, sem):
        blk = ci // _CB
        co = (ci % _CB) * _C

        @pl.when(co == 0)
        def _():
            stage_idx(blk)

        b = blk % 2
        pltpu.async_copy(z_sp.at[idx_s.at[b, pl.ds(co, _C)]], rs, sem)
        pltpu.async_copy(z_sp.at[idx_d.at[b, pl.ds(co, _C)]], rd, sem)

    def wait(rs, rd, sem):
        pltpu.make_async_copy(z_sp.at[idx_s.at[0, pl.ds(0, _C)]], rs,
                              sem).wait()
        pltpu.make_async_copy(z_sp.at[idx_d.at[0, pl.ds(0, _C)]], rd,
                              sem).wait()

    def compute(ci, rs, rd):
        def group_body(g, carry2):
            lane = lax.iota(jnp.int32, _L)
            perms = [(lane + s) % _L for s in (1, 2, 4, 8)]
            gvec = jnp.zeros((_L,), jnp.float32)
            for j in range(_L):
                e = g * _L + j
                acc = jnp.zeros((_L,), jnp.float32)
                for k in range(_D // (2 * _L)):
                    p = (rs[e, pl.ds(k * 2 * _L, 2 * _L)]
                         * rd[e, pl.ds(k * 2 * _L, 2 * _L)])
                    p0, p1 = plsc.unpack(
                        p, format=plsc.PackFormat.INTERLEAVED,
                        preferred_element_type=jnp.float32)
                    acc += p0 + p1
                for pm in perms:
                    acc = acc + lax.gather(
                        acc, pm[:, None],
                        lax.GatherDimensionNumbers(
                            offset_dims=(), collapsed_slice_dims=(0,),
                            start_index_map=(0,)),
                        slice_sizes=(1,),
                        mode=lax.GatherScatterMode.PROMISE_IN_BOUNDS)
                gvec = jnp.where(lane == j, acc, gvec)
            dots_v[pl.ds(g * _L, _L)] = gvec
            return carry2

        lax.fori_loop(0, _C // _L, group_body, 0)
        pltpu.sync_copy(dots_v, out_hbm.at[pl.ds(base_w + ci * _C, _C)])

    bufs = ((rows_s.at[0], rows_d.at[0], sem0),
            (rows_s.at[1], rows_d.at[1], sem1))

    issue(0, *bufs[0])

    def pair_body(i, carry):
        for b in range(2):
            ci = 2 * i + b
            rs, rd, sem = bufs[b]
            nrs, nrd, nsem = bufs[1 - b]

            @pl.when(ci < _NCHUNK)
            def _():
                wait(rs, rd, sem)

                @pl.when(ci + 1 < _NCHUNK)
                def _():
                    issue(ci + 1, nrs, nrd, nsem)

                compute(ci, rs, rd)
        return carry

    lax.fori_loop(0, (_NCHUNK + 1) // 2, pair_body, 0)


_sc_dots = functools.partial(
    pl.kernel,
    out_type=jax.ShapeDtypeStruct((_E2,), jnp.float32),
    mesh=plsc.VectorSubcoreMesh(core_axis_name="c", subcore_axis_name="s"),
    scratch_types=[
        pltpu.VMEM_SHARED((_N_NODES, _D), jnp.bfloat16),
        pltpu.VMEM((2, _BE), jnp.int32),
        pltpu.VMEM((2, _BE), jnp.int32),
        pltpu.VMEM((2, _C, _D), jnp.bfloat16),
        pltpu.VMEM((2, _C, _D), jnp.bfloat16),
        pltpu.VMEM((_C,), jnp.float32),
        pltpu.VMEM((_L, _L), jnp.float32),
        pltpu.SemaphoreType.DMA,
        pltpu.SemaphoreType.DMA,
    ],
    compiler_params=pltpu.CompilerParams(use_tc_tiling_on_sc=False,
                                         needs_layout_passes=False),
)(_sc_dots_body)


def _bce_body(x_ref, o_ref):
    x = x_ref[...]
    p = x[:_E // 128]
    n = x[_E // 128:]
    s = jnp.sum(jnp.maximum(-p, 0.0) + jnp.log1p(jnp.exp(-jnp.abs(p))))
    t = jnp.sum(jnp.maximum(n, 0.0) + jnp.log1p(jnp.exp(-jnp.abs(n))))
    o_ref[0, 0] = s / _E + t / _E


_bce = pl.pallas_call(
    _bce_body,
    out_shape=jax.ShapeDtypeStruct((1, 1), jnp.float32),
    out_specs=pl.BlockSpec(memory_space=pltpu.SMEM),
)


def kernel(z, pos_edge_index, neg_edge_index):
    dots = _sc_dots(z.astype(jnp.bfloat16),
                    pos_edge_index[0], pos_edge_index[1],
                    neg_edge_index[0], neg_edge_index[1])
    out = _bce(dots.reshape(_E2 // 128, 128))
    return out[0, 0]


# speedup vs baseline: 1.0469x; 1.0469x over previous
import functools

import jax
import jax.numpy as jnp
from jax import lax
from jax.experimental import pallas as pl
from jax.experimental.pallas import tpu as pltpu
from jax.experimental.pallas import tpu_sc as plsc

_N_NODES = 10000
_D = 256
_E = 160000
_E2 = 2 * _E
_NC, _NS, _L = 2, 16, 16
_NW = _NC * _NS
_EW = _E2 // _NW
_C = 80
_NCHUNK = _EW // _C
_DW = _D // 2
_CB = 25
_BE = _CB * _C


def _sc_dots_body(z_hbm, pos_hbm, neg_hbm, out_hbm,
                  z_sp, idx_s, idx_d, rows_s, rows_d, dots_v, red_v,
                  sem0, sem1):
    sid = lax.axis_index("s")
    wid = sid * _NC + lax.axis_index("c")
    base_w = wid * _EW

    rpt = _N_NODES // _NS
    pltpu.sync_copy(z_hbm.at[pl.ds(sid * rpt, rpt)],
                    z_sp.at[pl.ds(sid * rpt, rpt)])
    plsc.subcore_barrier()

    def stage_idx(blk):
        b = blk % 2

        @pl.when(wid < _NW // 2)
        def _():
            off = base_w + blk * _BE
            pltpu.sync_copy(pos_hbm.at[0, pl.ds(off, _BE)], idx_s.at[b])
            pltpu.sync_copy(pos_hbm.at[1, pl.ds(off, _BE)], idx_d.at[b])

        @pl.when(wid >= _NW // 2)
        def _():
            off = base_w - _E + blk * _BE
            pltpu.sync_copy(neg_hbm.at[0, pl.ds(off, _BE)], idx_s.at[b])
            pltpu.sync_copy(neg_hbm.at[1, pl.ds(off, _BE)], idx_d.at[b])

    def issue(ci, rs, rd, sem):
        blk = ci // _CB
        co = (ci % _CB) * _C

        @pl.when(co == 0)
        def _():
            stage_idx(blk)

        b = blk % 2
        pltpu.async_copy(z_sp.at[idx_s.at[b, pl.ds(co, _C)]], rs, sem)
        pltpu.async_copy(z_sp.at[idx_d.at[b, pl.ds(co, _C)]], rd, sem)

    def wait(rs, rd, sem):
        pltpu.make_async_copy(z_sp.at[idx_s.at[0, pl.ds(0, _C)]], rs,
                              sem).wait()
        pltpu.make_async_copy(z_sp.at[idx_d.at[0, pl.ds(0, _C)]], rd,
                              sem).wait()

    def compute(ci, rs, rd):
        def group_body(g, carry2):
            lane = lax.iota(jnp.int32, _L)
            perms = [(lane + s) % _L for s in (1, 2, 4, 8)]
            gvec = jnp.zeros((_L,), jnp.float32)
            for j in range(_L):
                e = g * _L + j
                acc = jnp.zeros((_L,), jnp.float32)
                for k in range(_D // (2 * _L)):
                    p = (rs[e, pl.ds(k * 2 * _L, 2 * _L)]
                         * rd[e, pl.ds(k * 2 * _L, 2 * _L)])
                    p0, p1 = plsc.unpack(
                        p, format=plsc.PackFormat.INTERLEAVED,
                        preferred_element_type=jnp.float32)
                    acc += p0 + p1
                for pm in perms:
                    acc = acc + lax.gather(
                        acc, pm[:, None],
                        lax.GatherDimensionNumbers(
                            offset_dims=(), collapsed_slice_dims=(0,),
                            start_index_map=(0,)),
                        slice_sizes=(1,),
                        mode=lax.GatherScatterMode.PROMISE_IN_BOUNDS)
                gvec = jnp.where(lane == j, acc, gvec)
            dots_v[pl.ds(g * _L, _L)] = gvec
            return carry2

        lax.fori_loop(0, _C // _L, group_body, 0)
        pltpu.sync_copy(dots_v, out_hbm.at[pl.ds(base_w + ci * _C, _C)])

    bufs = ((rows_s.at[0], rows_d.at[0], sem0),
            (rows_s.at[1], rows_d.at[1], sem1))

    issue(0, *bufs[0])

    def pair_body(i, carry):
        for b in range(2):
            ci = 2 * i + b
            rs, rd, sem = bufs[b]
            nrs, nrd, nsem = bufs[1 - b]

            @pl.when(ci < _NCHUNK)
            def _():
                wait(rs, rd, sem)

                @pl.when(ci + 1 < _NCHUNK)
                def _():
                    issue(ci + 1, nrs, nrd, nsem)

                compute(ci, rs, rd)
        return carry

    lax.fori_loop(0, (_NCHUNK + 1) // 2, pair_body, 0)


_sc_dots = functools.partial(
    pl.kernel,
    out_type=jax.ShapeDtypeStruct((_E2,), jnp.float32),
    mesh=plsc.VectorSubcoreMesh(core_axis_name="c", subcore_axis_name="s"),
    scratch_types=[
        pltpu.VMEM_SHARED((_N_NODES, _D), jnp.bfloat16),
        pltpu.VMEM((2, _BE), jnp.int32),
        pltpu.VMEM((2, _BE), jnp.int32),
        pltpu.VMEM((2, _C, _D), jnp.bfloat16),
        pltpu.VMEM((2, _C, _D), jnp.bfloat16),
        pltpu.VMEM((_C,), jnp.float32),
        pltpu.VMEM((_L, _L), jnp.float32),
        pltpu.SemaphoreType.DMA,
        pltpu.SemaphoreType.DMA,
    ],
    compiler_params=pltpu.CompilerParams(use_tc_tiling_on_sc=False,
                                         needs_layout_passes=False),
)(_sc_dots_body)


def _bce_body(x_ref, o_ref):
    x = x_ref[...]
    p = x[:_E // 128]
    n = x[_E // 128:]
    s = jnp.sum(jnp.maximum(-p, 0.0) + jnp.log1p(jnp.exp(-jnp.abs(p))))
    t = jnp.sum(jnp.maximum(n, 0.0) + jnp.log1p(jnp.exp(-jnp.abs(n))))
    o_ref[0, 0] = s / _E + t / _E


_bce = pl.pallas_call(
    _bce_body,
    out_shape=jax.ShapeDtypeStruct((1, 1), jnp.float32),
    out_specs=pl.BlockSpec(memory_space=pltpu.SMEM),
)


def kernel(z, pos_edge_index, neg_edge_index):
    dots = _sc_dots(z.astype(jnp.bfloat16), pos_edge_index, neg_edge_index)
    out = _bce(dots.reshape(_E2 // 128, 128))
    return out[0, 0]


# speedup vs baseline: 1.0501x; 1.0031x over previous
import functools

import jax
import jax.numpy as jnp
from jax import lax
from jax.experimental import pallas as pl
from jax.experimental.pallas import tpu as pltpu
from jax.experimental.pallas import tpu_sc as plsc

_N_NODES = 10000
_D = 256
_E = 160000
_E2 = 2 * _E
_NC, _NS, _L = 2, 16, 16
_NW = _NC * _NS
_EW = _E2 // _NW
_C = 80
_NCHUNK = _EW // _C
_CB = 25
_BE = _CB * _C


def _sc_dots_body(z_hbm, pos_hbm, neg_hbm, out_hbm,
                  z_sp, idx_s, idx_d, rows_s, rows_d, dots_v,
                  sem0, sem1):
    sid = lax.axis_index("s")
    wid = sid * _NC + lax.axis_index("c")
    base_w = wid * _EW

    rpt = _N_NODES // _NS
    pltpu.sync_copy(z_hbm.at[pl.ds(sid * rpt, rpt)],
                    z_sp.at[pl.ds(sid * rpt, rpt)])
    plsc.subcore_barrier()

    def stage_idx(blk):
        b = blk % 2

        @pl.when(wid < _NW // 2)
        def _():
            off = base_w + blk * _BE
            pltpu.sync_copy(pos_hbm.at[0, pl.ds(off, _BE)], idx_s.at[b])
            pltpu.sync_copy(pos_hbm.at[1, pl.ds(off, _BE)], idx_d.at[b])

        @pl.when(wid >= _NW // 2)
        def _():
            off = base_w - _E + blk * _BE
            pltpu.sync_copy(neg_hbm.at[0, pl.ds(off, _BE)], idx_s.at[b])
            pltpu.sync_copy(neg_hbm.at[1, pl.ds(off, _BE)], idx_d.at[b])

    def issue(ci, rs, rd, sem):
        blk = ci // _CB
        co = (ci % _CB) * _C

        @pl.when(co == 0)
        def _():
            stage_idx(blk)

        b = blk % 2
        pltpu.async_copy(z_sp.at[idx_s.at[b, pl.ds(co, _C)]], rs, sem)
        pltpu.async_copy(z_sp.at[idx_d.at[b, pl.ds(co, _C)]], rd, sem)

    def wait(rs, rd, sem):
        pltpu.make_async_copy(z_sp.at[idx_s.at[0, pl.ds(0, _C)]], rs,
                              sem).wait()
        pltpu.make_async_copy(z_sp.at[idx_d.at[0, pl.ds(0, _C)]], rd,
                              sem).wait()

    def compute(ci, rs, rd):
        def group_body(g, carry2):
            lane = lax.iota(jnp.int32, _L)
            perms = [(lane + s) % _L for s in (1, 2, 4, 8)]
            gvec = jnp.zeros((_L,), jnp.float32)
            for j in range(_L):
                e = g * _L + j
                acc = jnp.zeros((_L,), jnp.float32)
                for k in range(_D // (2 * _L)):
                    p = (rs[e, pl.ds(k * 2 * _L, 2 * _L)]
                         * rd[e, pl.ds(k * 2 * _L, 2 * _L)])
                    p0, p1 = plsc.unpack(
                        p, format=plsc.PackFormat.INTERLEAVED,
                        preferred_element_type=jnp.float32)
                    acc += p0 + p1
                for pm in perms:
                    acc = acc + lax.gather(
                        acc, pm[:, None],
                        lax.GatherDimensionNumbers(
                            offset_dims=(), collapsed_slice_dims=(0,),
                            start_index_map=(0,)),
                        slice_sizes=(1,),
                        mode=lax.GatherScatterMode.PROMISE_IN_BOUNDS)
                gvec = jnp.where(lane == j, acc, gvec)
            dots_v[pl.ds(g * _L, _L)] = gvec
            return carry2

        lax.fori_loop(0, _C // _L, group_body, 0)
        pltpu.sync_copy(dots_v, out_hbm.at[pl.ds(base_w + ci * _C, _C)])

    bufs = ((rows_s.at[0], rows_d.at[0], sem0),
            (rows_s.at[1], rows_d.at[1], sem1))

    issue(0, *bufs[0])

    def pair_body(i, carry):
        for b in range(2):
            ci = 2 * i + b
            rs, rd, sem = bufs[b]
            nrs, nrd, nsem = bufs[1 - b]

            @pl.when(ci < _NCHUNK)
            def _():
                wait(rs, rd, sem)

                @pl.when(ci + 1 < _NCHUNK)
                def _():
                    issue(ci + 1, nrs, nrd, nsem)

                compute(ci, rs, rd)
        return carry

    lax.fori_loop(0, (_NCHUNK + 1) // 2, pair_body, 0)


_sc_dots = functools.partial(
    pl.kernel,
    out_type=jax.ShapeDtypeStruct((_E2,), jnp.float32),
    mesh=plsc.VectorSubcoreMesh(core_axis_name="c", subcore_axis_name="s"),
    scratch_types=[
        pltpu.VMEM_SHARED((_N_NODES, _D), jnp.bfloat16),
        pltpu.VMEM((2, _BE), jnp.int32),
        pltpu.VMEM((2, _BE), jnp.int32),
        pltpu.VMEM((2, _C, _D), jnp.bfloat16),
        pltpu.VMEM((2, _C, _D), jnp.bfloat16),
        pltpu.VMEM((_C,), jnp.float32),
        pltpu.SemaphoreType.DMA,
        pltpu.SemaphoreType.DMA,
    ],
    compiler_params=pltpu.CompilerParams(use_tc_tiling_on_sc=False,
                                         needs_layout_passes=False),
)(_sc_dots_body)


def _bce_body(x_ref, o_ref):
    x = x_ref[...]
    p = x[:_E // 128]
    n = x[_E // 128:]
    s = jnp.sum(jnp.maximum(-p, 0.0) + jnp.log1p(jnp.exp(-jnp.abs(p))))
    t = jnp.sum(jnp.maximum(n, 0.0) + jnp.log1p(jnp.exp(-jnp.abs(n))))
    o_ref[0, 0] = s / _E + t / _E


_bce = pl.pallas_call(
    _bce_body,
    out_shape=jax.ShapeDtypeStruct((1, 1), jnp.float32),
    out_specs=pl.BlockSpec(memory_space=pltpu.SMEM),
)


def kernel(z, pos_edge_index, neg_edge_index):
    dots = _sc_dots(z.astype(jnp.bfloat16), pos_edge_index, neg_edge_index)
    out = _bce(dots.reshape(_E2 // 128, 128))
    return out[0, 0]
